# baseline (speedup 1.0000x reference)
import functools

import jax
import jax.numpy as jnp
from jax import lax
from jax.experimental import pallas as pl
from jax.experimental.pallas import tpu as pltpu
from jax.experimental.pallas import tpu_sc as plsc

B = 4096
F = 26
E = 16
N = 64
V = 1000000
TEMP = 0.5
LAMBDA_C = 0.5

BF = B * F
NW = 32
RPW = BF // NW
CH = 128
NCH = RPW // CH

VB = 4096
TGRID = -(-V // VB)


def _tx_body(x0_r, x1_r, x2_r, o0_r, o1_r, o2_r):
    for x_r, o_r in ((x0_r, o0_r), (x1_r, o1_r), (x2_r, o2_r)):
        x3 = jnp.transpose(x_r[...]).reshape(VB // 8, 8, E)
        o_r[...] = jnp.concatenate([x3[:, j, :] for j in range(8)], axis=1)


_tx_call = pl.pallas_call(
    _tx_body,
    grid=(TGRID,),
    in_specs=[pl.BlockSpec((E, VB), lambda i: (0, i))] * 3,
    out_specs=[pl.BlockSpec((VB // 8, 128), lambda i: (i, 0))] * 3,
    out_shape=[jax.ShapeDtypeStruct((V // 8, 128), jnp.float32)] * 3,
)


def _sc_gather_body(fid, lin128, e0, e1, e2,
                    lin_o, ev0_o, ev1_o, ev2_o,
                    idx_v, rows_v, lin_v, rowid_v, wide_v, sem):
    wid = lax.axis_index("s") * 2 + lax.axis_index("c")
    base = wid * RPW
    pltpu.sync_copy(fid.at[wid], idx_v)
    iota16 = lax.iota(jnp.int32, 16)

    def prep(c, _):
        for j in range(CH // 16):
            v = idx_v[c, pl.ds(j * 16, 16)]
            rowid_v[c, pl.ds(j * 16, 16)] = lax.shift_right_logical(v, 3)
        return 0
    lax.fori_loop(0, NCH, prep, 0)

    for tab, out in ((e0, ev0_o), (e1, ev1_o), (e2, ev2_o)):
        def gather_chunk(c, _, tab=tab):
            pltpu.make_async_copy(tab.at[rowid_v.at[c]],
                                  wide_v.at[c % 2], sem).wait()

            @pl.when(c + 1 < NCH)
            def _():
                pltpu.async_copy(tab.at[rowid_v.at[c + 1]],
                                 wide_v.at[(c + 1) % 2], sem)

            for j in range(CH // 16):
                v = idx_v[c, pl.ds(j * 16, 16)]
                csrc = lax.shift_left(lax.bitwise_and(v, 7), 4)
                rsrc = iota16 + j * 16
                pvec = c * CH + j * 16 + iota16
                rdst = lax.shift_right_logical(pvec, 3)
                cdst = lax.shift_left(lax.bitwise_and(pvec, 7), 4)
                for e in range(E):
                    vals = plsc.load_gather(wide_v.at[c % 2], [rsrc, csrc + e])
                    plsc.store_scatter(rows_v, [rdst, cdst + e], vals)
            return 0
        pltpu.async_copy(tab.at[rowid_v.at[0]], wide_v.at[0], sem)
        lax.fori_loop(0, NCH, gather_chunk, 0)
        pltpu.sync_copy(rows_v, out.at[pl.ds(wid * (RPW * E // 128),
                                             RPW * E // 128)])

    def gather_lin(c, _):
        for j in range(CH // 16):
            v = idx_v[c, pl.ds(j * 16, 16)]
            rowid_v[0, pl.ds(j * 16, 16)] = lax.shift_right_logical(v, 7)
        pltpu.async_copy(lin128.at[rowid_v.at[0]], wide_v.at[0], sem).wait()
        for j in range(CH // 16):
            v = idx_v[c, pl.ds(j * 16, 16)]
            col = lax.bitwise_and(v, 127)
            lin_v[pl.ds(c * CH + j * 16, 16)] = plsc.load_gather(
                wide_v.at[0], [iota16 + j * 16, col])
        return 0
    lax.fori_loop(0, NCH, gather_lin, 0)
    pltpu.sync_copy(lin_v, lin_o.at[pl.ds(base, RPW)])


@functools.cache
def _sc_gather():
    return pl.kernel(
        _sc_gather_body,
        out_type=[
            jax.ShapeDtypeStruct((BF,), jnp.float32),
            jax.ShapeDtypeStruct((BF * E // 128, 128), jnp.float32),
            jax.ShapeDtypeStruct((BF * E // 128, 128), jnp.float32),
            jax.ShapeDtypeStruct((BF * E // 128, 128), jnp.float32),
        ],
        mesh=plsc.VectorSubcoreMesh(core_axis_name="c", subcore_axis_name="s"),
        scratch_types=[
            pltpu.VMEM((NCH, CH), jnp.int32),
            pltpu.VMEM((RPW * E // 128, 128), jnp.float32),
            pltpu.VMEM((RPW,), jnp.float32),
            pltpu.VMEM((NCH, CH), jnp.int32),
            pltpu.VMEM((2, CH, 128), jnp.float32),
            pltpu.SemaphoreType.DMA,
        ],
        compiler_params=pltpu.CompilerParams(use_tc_tiling_on_sc=True,
                                             needs_layout_passes=False),
    )


BT = 512


def _tc_body(ev0_r, ev1_r, ev2_r, linr_r,
             we0_r, we1_r, we2_r, s_r, wp2_r, wp3_r,
             w1a_r, w1b_r, w1c_r, b1_r, w2_r, b2_r, w3_r, b3_r,
             wo_r, c0_r, o_r):
    f32 = jnp.float32
    s_mat = s_r[...]
    x1 = jnp.dot(ev0_r[...], we0_r[...], preferred_element_type=f32)
    ev1 = ev1_r[...]
    h2 = jnp.dot(ev1, we1_r[...], preferred_element_type=f32)
    p2 = (jnp.dot(h2 * h2, s_mat, preferred_element_type=f32)
          - LAMBDA_C * jnp.dot(ev1 * ev1, wp2_r[...], preferred_element_type=f32))
    ev2 = ev2_r[...]
    h3 = jnp.dot(ev2, we2_r[...], preferred_element_type=f32)
    p3 = (jnp.dot(h3 * h3 * h3, s_mat, preferred_element_type=f32)
          - LAMBDA_C * jnp.dot(ev2 * ev2 * ev2, wp3_r[...], preferred_element_type=f32))
    h = (jnp.dot(x1, w1a_r[...], preferred_element_type=f32)
         + jnp.dot(p2, w1b_r[...], preferred_element_type=f32)
         + jnp.dot(p3, w1c_r[...], preferred_element_type=f32)
         + b1_r[...])
    h = jnp.maximum(h, 0.0)
    h = jnp.maximum(jnp.dot(h, w2_r[...], preferred_element_type=f32) + b2_r[...], 0.0)
    h = jnp.maximum(jnp.dot(h, w3_r[...], preferred_element_type=f32) + b3_r[...], 0.0)
    y = jnp.sum(h * wo_r[...], axis=1)
    lin = jnp.sum(linr_r[...], axis=1)
    o_r[...] = y + lin + c0_r[0, 0]


def _full(shape):
    return pl.BlockSpec(shape, lambda i: (0, 0))


_tc_call = pl.pallas_call(
    _tc_body,
    grid=(B // BT,),
    in_specs=[
        pl.BlockSpec((BT, F * E), lambda i: (i, 0)),
        pl.BlockSpec((BT, F * E), lambda i: (i, 0)),
        pl.BlockSpec((BT, F * E), lambda i: (i, 0)),
        pl.BlockSpec((BT, F), lambda i: (i, 0)),
        _full((F * E, N * E)),
        _full((F * E, N * E)),
        _full((F * E, N * E)),
        _full((N * E, N)),
        _full((F * E, N)),
        _full((F * E, N)),
        _full((N * E, 400)),
        _full((N, 400)),
        _full((N, 400)),
        _full((1, 400)),
        _full((400, 400)),
        _full((1, 400)),
        _full((400, 400)),
        _full((1, 400)),
        _full((1, 400)),
        _full((1, 1)),
    ],
    out_specs=pl.BlockSpec((BT,), lambda i: (i,)),
    out_shape=jax.ShapeDtypeStruct((B,), jnp.float32),
)


def _select_wt(sl, hw):
    y = jax.nn.softmax(sl / TEMP, axis=-1)
    y_hard = (y == jnp.max(y, axis=-1, keepdims=True)).astype(y.dtype)
    c = ((y_hard - y) + y)[..., 0]
    return c * hw


def kernel(feature_id, lin_w, lin_b, emb0, emb1, emb2, sl0, sl1, sl2,
           hw0, hw1, hw2, w1, b1, w2, b2, w3, b3, wo, bo):
    fid = feature_id.astype(jnp.int32).reshape(NW, NCH, CH)
    lin128 = jnp.pad(lin_w[:, 0], (0, 64)).reshape(V // 128 + 1, 128)
    e0_8, e1_8, e2_8 = _tx_call(emb0.T, emb1.T, emb2.T)
    lin_g, ev0, ev1, ev2 = _sc_gather()(fid, lin128, e0_8, e1_8, e2_8)

    eye_e = jnp.eye(E, dtype=jnp.float32)
    wts = [_select_wt(sl, hw) for sl, hw in ((sl0, hw0), (sl1, hw1), (sl2, hw2))]
    wes = [jnp.einsum('fn,ec->fenc', wt, eye_e).reshape(F * E, N * E)
           for wt in wts]
    s_mat = jnp.kron(jnp.eye(N, dtype=jnp.float32),
                     jnp.ones((E, 1), dtype=jnp.float32))
    wp2 = jnp.repeat(wts[1] ** 2, E, axis=0)
    wp3 = jnp.repeat(wts[2] ** 3, E, axis=0)

    out = _tc_call(
        ev0.reshape(B, F * E), ev1.reshape(B, F * E), ev2.reshape(B, F * E),
        lin_g.reshape(B, F),
        wes[0], wes[1], wes[2], s_mat, wp2, wp3,
        w1[:N * E], w1[N * E:N * E + N], w1[N * E + N:],
        b1.reshape(1, 400), w2, b2.reshape(1, 400), w3, b3.reshape(1, 400),
        wo.reshape(1, 400), (lin_b[0] + bo[0]).reshape(1, 1),
    )
    return out


# speedup vs baseline: 1.1139x; 1.1139x over previous
import functools

import jax
import jax.numpy as jnp
from jax import lax
from jax.experimental import pallas as pl
from jax.experimental.pallas import tpu as pltpu
from jax.experimental.pallas import tpu_sc as plsc

B = 4096
F = 26
E = 16
N = 64
V = 1000000
TEMP = 0.5
LAMBDA_C = 0.5

BF = B * F
NW = 32
RPW = BF // NW
CH = 128
NCH = RPW // CH

VB = 4096
TGRID = -(-V // VB)


def _tx_body(x0_r, x1_r, x2_r, o0_r, o1_r, o2_r):
    for x_r, o_r in ((x0_r, o0_r), (x1_r, o1_r), (x2_r, o2_r)):
        x3 = jnp.transpose(x_r[...]).reshape(VB // 8, 8, E)
        for j in range(8):
            o_r[:, j * E:(j + 1) * E] = x3[:, j, :]


_tx_call = pl.pallas_call(
    _tx_body,
    grid=(TGRID,),
    in_specs=[pl.BlockSpec((E, VB), lambda i: (0, i))] * 3,
    out_specs=[pl.BlockSpec((VB // 8, 128), lambda i: (i, 0))] * 3,
    out_shape=[jax.ShapeDtypeStruct((V // 8, 128), jnp.float32)] * 3,
)


def _sc_gather_body(fid, lin128, e0, e1, e2,
                    lin_o, ev0_o, ev1_o, ev2_o,
                    idx_v, rows_v, lin_v, rowid_v, wide_v, sem):
    wid = lax.axis_index("s") * 2 + lax.axis_index("c")
    base = wid * RPW
    pltpu.sync_copy(fid.at[wid], idx_v)
    iota16 = lax.iota(jnp.int32, 16)

    def prep(c, _):
        for j in range(CH // 16):
            v = idx_v[c, pl.ds(j * 16, 16)]
            rowid_v[c, pl.ds(j * 16, 16)] = lax.shift_right_logical(v, 3)
        return 0
    lax.fori_loop(0, NCH, prep, 0)

    for tab, out in ((e0, ev0_o), (e1, ev1_o), (e2, ev2_o)):
        def gather_chunk(c, _, tab=tab):
            pltpu.make_async_copy(tab.at[rowid_v.at[c]],
                                  wide_v.at[c % 2], sem).wait()

            @pl.when(c + 1 < NCH)
            def _():
                pltpu.async_copy(tab.at[rowid_v.at[c + 1]],
                                 wide_v.at[(c + 1) % 2], sem)

            for j in range(CH // 16):
                v = idx_v[c, pl.ds(j * 16, 16)]
                csrc = lax.shift_left(lax.bitwise_and(v, 7), 4)
                rsrc = iota16 + j * 16
                pvec = c * CH + j * 16 + iota16
                rdst = lax.shift_right_logical(pvec, 3)
                cdst = lax.shift_left(lax.bitwise_and(pvec, 7), 4)
                for e in range(E):
                    vals = plsc.load_gather(wide_v.at[c % 2], [rsrc, csrc + e])
                    plsc.store_scatter(rows_v, [rdst, cdst + e], vals)
            return 0
        pltpu.async_copy(tab.at[rowid_v.at[0]], wide_v.at[0], sem)
        lax.fori_loop(0, NCH, gather_chunk, 0)
        pltpu.sync_copy(rows_v, out.at[pl.ds(wid * (RPW * E // 128),
                                             RPW * E // 128)])

    def gather_lin(c, _):
        for j in range(CH // 16):
            v = idx_v[c, pl.ds(j * 16, 16)]
            rowid_v[0, pl.ds(j * 16, 16)] = lax.shift_right_logical(v, 7)
        pltpu.async_copy(lin128.at[rowid_v.at[0]], wide_v.at[0], sem).wait()
        for j in range(CH // 16):
            v = idx_v[c, pl.ds(j * 16, 16)]
            col = lax.bitwise_and(v, 127)
            lin_v[pl.ds(c * CH + j * 16, 16)] = plsc.load_gather(
                wide_v.at[0], [iota16 + j * 16, col])
        return 0
    lax.fori_loop(0, NCH, gather_lin, 0)
    pltpu.sync_copy(lin_v, lin_o.at[pl.ds(base, RPW)])


@functools.cache
def _sc_gather():
    return pl.kernel(
        _sc_gather_body,
        out_type=[
            jax.ShapeDtypeStruct((BF,), jnp.float32),
            jax.ShapeDtypeStruct((BF * E // 128, 128), jnp.float32),
            jax.ShapeDtypeStruct((BF * E // 128, 128), jnp.float32),
            jax.ShapeDtypeStruct((BF * E // 128, 128), jnp.float32),
        ],
        mesh=plsc.VectorSubcoreMesh(core_axis_name="c", subcore_axis_name="s"),
        scratch_types=[
            pltpu.VMEM((NCH, CH), jnp.int32),
            pltpu.VMEM((RPW * E // 128, 128), jnp.float32),
            pltpu.VMEM((RPW,), jnp.float32),
            pltpu.VMEM((NCH, CH), jnp.int32),
            pltpu.VMEM((2, CH, 128), jnp.float32),
            pltpu.SemaphoreType.DMA,
        ],
        compiler_params=pltpu.CompilerParams(use_tc_tiling_on_sc=True,
                                             needs_layout_passes=False),
    )


BT = 512


def _tc_body(ev0_r, ev1_r, ev2_r, linr_r,
             we0_r, we1_r, we2_r, s_r, wp2_r, wp3_r,
             w1a_r, w1b_r, w1c_r, b1_r, w2_r, b2_r, w3_r, b3_r,
             wo_r, c0_r, o_r):
    f32 = jnp.float32
    s_mat = s_r[...]
    x1 = jnp.dot(ev0_r[...], we0_r[...], preferred_element_type=f32)
    ev1 = ev1_r[...]
    h2 = jnp.dot(ev1, we1_r[...], preferred_element_type=f32)
    p2 = (jnp.dot(h2 * h2, s_mat, preferred_element_type=f32)
          - LAMBDA_C * jnp.dot(ev1 * ev1, wp2_r[...], preferred_element_type=f32))
    ev2 = ev2_r[...]
    h3 = jnp.dot(ev2, we2_r[...], preferred_element_type=f32)
    p3 = (jnp.dot(h3 * h3 * h3, s_mat, preferred_element_type=f32)
          - LAMBDA_C * jnp.dot(ev2 * ev2 * ev2, wp3_r[...], preferred_element_type=f32))
    h = (jnp.dot(x1, w1a_r[...], preferred_element_type=f32)
         + jnp.dot(p2, w1b_r[...], preferred_element_type=f32)
         + jnp.dot(p3, w1c_r[...], preferred_element_type=f32)
         + b1_r[...])
    h = jnp.maximum(h, 0.0)
    h = jnp.maximum(jnp.dot(h, w2_r[...], preferred_element_type=f32) + b2_r[...], 0.0)
    h = jnp.maximum(jnp.dot(h, w3_r[...], preferred_element_type=f32) + b3_r[...], 0.0)
    y = jnp.sum(h * wo_r[...], axis=1)
    lin = jnp.sum(linr_r[...], axis=1)
    o_r[...] = y + lin + c0_r[0, 0]


def _full(shape):
    return pl.BlockSpec(shape, lambda i: (0, 0))


_tc_call = pl.pallas_call(
    _tc_body,
    grid=(B // BT,),
    in_specs=[
        pl.BlockSpec((BT, F * E), lambda i: (i, 0)),
        pl.BlockSpec((BT, F * E), lambda i: (i, 0)),
        pl.BlockSpec((BT, F * E), lambda i: (i, 0)),
        pl.BlockSpec((BT, F), lambda i: (i, 0)),
        _full((F * E, N * E)),
        _full((F * E, N * E)),
        _full((F * E, N * E)),
        _full((N * E, N)),
        _full((F * E, N)),
        _full((F * E, N)),
        _full((N * E, 400)),
        _full((N, 400)),
        _full((N, 400)),
        _full((1, 400)),
        _full((400, 400)),
        _full((1, 400)),
        _full((400, 400)),
        _full((1, 400)),
        _full((1, 400)),
        _full((1, 1)),
    ],
    out_specs=pl.BlockSpec((BT,), lambda i: (i,)),
    out_shape=jax.ShapeDtypeStruct((B,), jnp.float32),
)


def _select_wt(sl, hw):
    y = jax.nn.softmax(sl / TEMP, axis=-1)
    y_hard = (y == jnp.max(y, axis=-1, keepdims=True)).astype(y.dtype)
    c = ((y_hard - y) + y)[..., 0]
    return c * hw


def kernel(feature_id, lin_w, lin_b, emb0, emb1, emb2, sl0, sl1, sl2,
           hw0, hw1, hw2, w1, b1, w2, b2, w3, b3, wo, bo):
    fid = feature_id.astype(jnp.int32).reshape(NW, NCH, CH)
    lin128 = jnp.pad(lin_w[:, 0], (0, 64)).reshape(V // 128 + 1, 128)
    e0_8, e1_8, e2_8 = _tx_call(emb0.T, emb1.T, emb2.T)
    lin_g, ev0, ev1, ev2 = _sc_gather()(fid, lin128, e0_8, e1_8, e2_8)

    eye_e = jnp.eye(E, dtype=jnp.float32)
    wts = [_select_wt(sl, hw) for sl, hw in ((sl0, hw0), (sl1, hw1), (sl2, hw2))]
    wes = [jnp.einsum('fn,ec->fenc', wt, eye_e).reshape(F * E, N * E)
           for wt in wts]
    s_mat = jnp.kron(jnp.eye(N, dtype=jnp.float32),
                     jnp.ones((E, 1), dtype=jnp.float32))
    wp2 = jnp.repeat(wts[1] ** 2, E, axis=0)
    wp3 = jnp.repeat(wts[2] ** 3, E, axis=0)

    out = _tc_call(
        ev0.reshape(B, F * E), ev1.reshape(B, F * E), ev2.reshape(B, F * E),
        lin_g.reshape(B, F),
        wes[0], wes[1], wes[2], s_mat, wp2, wp3,
        w1[:N * E], w1[N * E:N * E + N], w1[N * E + N:],
        b1.reshape(1, 400), w2, b2.reshape(1, 400), w3, b3.reshape(1, 400),
        wo.reshape(1, 400), (lin_b[0] + bo[0]).reshape(1, 1),
    )
    return out
